# baseline (speedup 1.0000x reference)
import functools

import jax
import jax.numpy as jnp
import numpy as np
from jax import lax
from jax.experimental import pallas as pl
from jax.experimental.pallas import tpu as pltpu
from jax.experimental.pallas import tpu_sc as plsc

_N_STATES = 16
_N_SYMBOLS = 8
_SEQ_LEN = 200
_LANES = 16
_NCHUNK = (_SEQ_LEN + _LANES - 1) // _LANES
_PADDED = _NCHUNK * _LANES
_ZROW = 33

_DFA = {0: {'0': 1, '1': 2, 'and(2,3)': 3}, 1: {'2': 3, 'not(0)': 0, '4': 5}, 2: {'or(1,5)': 4, '3': 2}, 3: {'5': 6, 'T': 0}, 4: {'6': 7, 'and(0,not(1))': 8}, 5: {'7': 9, '2': 5}, 6: {'or(and(0,1),2)': 10, '4': 6}, 7: {'1': 11, 'not(6)': 7}, 8: {'3': 12, '0': 8}, 9: {'5': 13, 'or(2,3)': 9}, 10: {'and(4,5)': 14, '6': 10}, 11: {'7': 15, '1': 11}, 12: {'0': 0, 'not(7)': 12}, 13: {'2': 1, '6': 13}, 14: {'or(0,not(4))': 2, '3': 14}, 15: {'T': 3}}

_TRANS = [(s, g, d) for s in sorted(_DFA.keys()) for g, d in _DFA[s].items()]

_INC = {}
for _t, (_s, _g, _d) in enumerate(_TRANS):
    _INC.setdefault(_d, []).append((_t, _s))
_N_GROUPS = max(len(v) for v in _INC.values())
_TID = np.full((_N_GROUPS, _N_STATES), _ZROW, np.int32)
_SRC = np.zeros((_N_GROUPS, _N_STATES), np.int32)
for _d in range(_N_STATES):
    for _j, (_t, _s) in enumerate(_INC[_d]):
        _TID[_j, _d] = _t
        _SRC[_j, _d] = _s


def _divide_args(guard):
    args = guard.split(',')
    out = []
    i = 0
    while i < len(args):
        a = args[i]
        while a.count('(') != a.count(')'):
            i += 1
            a = a + ',' + args[i]
        out.append(a)
        i += 1
    return out


def _eval_guard(guard, cols):
    if guard[0] == 'a':
        v = 1.0
        for a in _divide_args(guard[4:-1]):
            v = v * _eval_guard(a, cols)
        return v
    elif guard[0] == 'o':
        v = 0.0
        for a in _divide_args(guard[3:-1]):
            e = _eval_guard(a, cols)
            v = v + e - v * e
        return v
    elif guard[0] == 'n':
        return 1.0 - _eval_guard(guard[4:-1], cols)
    elif guard[0] == 'T':
        return jnp.ones_like(cols[0])
    else:
        return cols[int(guard)]


_GATHER_DNUMS = lax.GatherDimensionNumbers(
    offset_dims=(), collapsed_slice_dims=(0,), start_index_map=(0,))


def _vgather(x, idx):
    return lax.gather(x, idx[:, None], _GATHER_DNUMS, (1,),
                      mode=lax.GatherScatterMode.PROMISE_IN_BOUNDS)


def _sc_body(p_hbm, idx_hbm, out_hbm, p_v, idx_v, g_v, out_v):
    wid = lax.axis_index("s") * 2 + lax.axis_index("c")

    @pl.when(wid == 0)
    def _():
        pltpu.sync_copy(p_hbm, p_v.at[pl.ds(0, _SEQ_LEN)])
        pltpu.sync_copy(idx_hbm, idx_v)

        lane = lax.broadcasted_iota(jnp.int32, (_LANES,), 0)
        for c in range(_NCHUNK):
            ids = lane + (c * _LANES)
            cols = [plsc.load_gather(p_v, [ids, jnp.full((_LANES,), k, jnp.int32)])
                    for k in range(_N_SYMBOLS)]
            for t, (_, g, _2) in enumerate(_TRANS):
                g_v[t, pl.ds(c * _LANES, _LANES)] = _eval_guard(g, cols)
            g_v[_ZROW, pl.ds(c * _LANES, _LANES)] = jnp.zeros((_LANES,), jnp.float32)

        tids = [idx_v[j, :] for j in range(_N_GROUPS)]
        srcs = [idx_v[_N_GROUPS + j, :] for j in range(_N_GROUPS)]
        st0 = (lane == 0).astype(jnp.float32)

        unroll = 8

        def body(i, st):
            base = jnp.full((_LANES,), i * unroll, jnp.int32)
            for k in range(unroll):
                ii = base + k
                nxt = None
                for j in range(_N_GROUPS):
                    gj = plsc.load_gather(g_v, [tids[j], ii])
                    term = _vgather(st, srcs[j]) * gj
                    nxt = term if nxt is None else nxt + term
                st = nxt
            return st

        st = lax.fori_loop(0, _SEQ_LEN // unroll, body, st0)
        out_v[...] = st
        pltpu.sync_copy(out_v, out_hbm)


def kernel(symbols_prob):
    mesh = plsc.VectorSubcoreMesh(core_axis_name="c", subcore_axis_name="s")
    run = pl.kernel(
        _sc_body, mesh=mesh,
        out_type=jax.ShapeDtypeStruct((_N_STATES,), jnp.float32),
        compiler_params=pltpu.CompilerParams(
            needs_layout_passes=False,
            skip_device_barrier=True,
            disable_bounds_checks=True,
            disable_semaphore_checks=True,
        ),
        scratch_types=[
            pltpu.VMEM((_PADDED, _N_SYMBOLS), jnp.float32),
            pltpu.VMEM((2 * _N_GROUPS, _LANES), jnp.int32),
            pltpu.VMEM((_ZROW + 1, _PADDED), jnp.float32),
            pltpu.VMEM((_N_STATES,), jnp.float32),
        ],
    )
    idx_tab = jnp.asarray(np.concatenate([_TID, _SRC], axis=0))
    return run(symbols_prob, idx_tab)


# speedup vs baseline: 1.9520x; 1.9520x over previous
import jax
import jax.numpy as jnp
import numpy as np
from jax.experimental import pallas as pl
from jax.experimental.pallas import tpu as pltpu

_N_STATES = 16
_N_SYMBOLS = 8
_SEQ_LEN = 200

_DFA = {0: {'0': 1, '1': 2, 'and(2,3)': 3}, 1: {'2': 3, 'not(0)': 0, '4': 5}, 2: {'or(1,5)': 4, '3': 2}, 3: {'5': 6, 'T': 0}, 4: {'6': 7, 'and(0,not(1))': 8}, 5: {'7': 9, '2': 5}, 6: {'or(and(0,1),2)': 10, '4': 6}, 7: {'1': 11, 'not(6)': 7}, 8: {'3': 12, '0': 8}, 9: {'5': 13, 'or(2,3)': 9}, 10: {'and(4,5)': 14, '6': 10}, 11: {'7': 15, '1': 11}, 12: {'0': 0, 'not(7)': 12}, 13: {'2': 1, '6': 13}, 14: {'or(0,not(4))': 2, '3': 14}, 15: {'T': 3}}

_TRANS = [(s, g, d) for s in sorted(_DFA.keys()) for g, d in _DFA[s].items()]


def _divide_args(guard):
    args = guard.split(',')
    out = []
    i = 0
    while i < len(args):
        a = args[i]
        while a.count('(') != a.count(')'):
            i += 1
            a = a + ',' + args[i]
        out.append(a)
        i += 1
    return out


def _eval_guard(guard, cols):
    if guard[0] == 'a':
        v = 1.0
        for a in _divide_args(guard[4:-1]):
            v = v * _eval_guard(a, cols)
        return v
    elif guard[0] == 'o':
        v = 0.0
        for a in _divide_args(guard[3:-1]):
            e = _eval_guard(a, cols)
            v = v + e - v * e
        return v
    elif guard[0] == 'n':
        return 1.0 - _eval_guard(guard[4:-1], cols)
    elif guard[0] == 'T':
        return jnp.ones_like(cols[0])
    else:
        return cols[int(guard)]


_EDGE = {(d, s): t for t, (s, _, d) in enumerate(_TRANS)}


def _build_mats(p_block):
    L = p_block.shape[0]
    cols = [p_block[:, k:k + 1] for k in range(_N_SYMBOLS)]
    gvals = [_eval_guard(g, cols) for (_, g, _) in _TRANS]
    zero = jnp.zeros((L, 1), dtype=p_block.dtype)

    def stack2d(index_fn):
        rows = []
        for a in range(_N_STATES):
            row = [index_fn(a, b) for b in range(_N_STATES)]
            rows.append(jnp.concatenate(row, axis=1)[:, None, :])
        return jnp.concatenate(rows, axis=1)

    mats_ds = stack2d(lambda d, s: gvals[_EDGE[(d, s)]] if (d, s) in _EDGE else zero)
    mats_sd = stack2d(lambda s, d: gvals[_EDGE[(d, s)]] if (d, s) in _EDGE else zero)
    return mats_ds, mats_sd


def _fuzzy_kernel(p_ref, out_ref, ads_ref, asd_ref):
    p = p_ref[:, :].reshape(_SEQ_LEN // 2, 2, _N_SYMBOLS)
    p_even = p[:, 0, :]
    p_odd = p[:, 1, :]

    ads_ref[:, :, :], _ = _build_mats(p_even)
    _, asd = _build_mats(p_odd)
    asd_ref[:, :, :] = asd

    st0 = (jax.lax.broadcasted_iota(jnp.int32, (1, _N_STATES), 1) == 0
           ).astype(p_ref.dtype)

    unroll = 10

    def body(i, st):
        for k in range(unroll):
            a = ads_ref[i * unroll + k]
            mid = jnp.sum(a * st, axis=1, keepdims=True)
            b = asd_ref[i * unroll + k]
            st = jnp.sum(b * mid, axis=0, keepdims=True)
        return st

    st = jax.lax.fori_loop(0, _SEQ_LEN // 2 // unroll, body, st0)
    out_ref[:, :] = st


def kernel(symbols_prob):
    out = pl.pallas_call(
        _fuzzy_kernel,
        out_shape=jax.ShapeDtypeStruct((1, _N_STATES), symbols_prob.dtype),
        scratch_shapes=[
            pltpu.VMEM((_SEQ_LEN // 2, _N_STATES, _N_STATES), symbols_prob.dtype),
            pltpu.VMEM((_SEQ_LEN // 2, _N_STATES, _N_STATES), symbols_prob.dtype),
        ],
    )(symbols_prob)
    return out.reshape(_N_STATES)


# speedup vs baseline: 2.6797x; 1.3728x over previous
import jax
import jax.numpy as jnp
import numpy as np
from jax.experimental import pallas as pl
from jax.experimental.pallas import tpu as pltpu

_N_STATES = 16
_N_SYMBOLS = 8
_SEQ_LEN = 200

_DFA = {0: {'0': 1, '1': 2, 'and(2,3)': 3}, 1: {'2': 3, 'not(0)': 0, '4': 5}, 2: {'or(1,5)': 4, '3': 2}, 3: {'5': 6, 'T': 0}, 4: {'6': 7, 'and(0,not(1))': 8}, 5: {'7': 9, '2': 5}, 6: {'or(and(0,1),2)': 10, '4': 6}, 7: {'1': 11, 'not(6)': 7}, 8: {'3': 12, '0': 8}, 9: {'5': 13, 'or(2,3)': 9}, 10: {'and(4,5)': 14, '6': 10}, 11: {'7': 15, '1': 11}, 12: {'0': 0, 'not(7)': 12}, 13: {'2': 1, '6': 13}, 14: {'or(0,not(4))': 2, '3': 14}, 15: {'T': 3}}

_TRANS = [(s, g, d) for s in sorted(_DFA.keys()) for g, d in _DFA[s].items()]


def _divide_args(guard):
    args = guard.split(',')
    out = []
    i = 0
    while i < len(args):
        a = args[i]
        while a.count('(') != a.count(')'):
            i += 1
            a = a + ',' + args[i]
        out.append(a)
        i += 1
    return out


def _eval_guard(guard, cols):
    if guard[0] == 'a':
        v = 1.0
        for a in _divide_args(guard[4:-1]):
            v = v * _eval_guard(a, cols)
        return v
    elif guard[0] == 'o':
        v = 0.0
        for a in _divide_args(guard[3:-1]):
            e = _eval_guard(a, cols)
            v = v + e - v * e
        return v
    elif guard[0] == 'n':
        return 1.0 - _eval_guard(guard[4:-1], cols)
    elif guard[0] == 'T':
        return jnp.ones_like(cols[0])
    else:
        return cols[int(guard)]


_EDGE = {(d, s): t for t, (s, _, d) in enumerate(_TRANS)}


def _build_mats(p_block):
    L = p_block.shape[0]
    cols = [p_block[:, k:k + 1] for k in range(_N_SYMBOLS)]
    gvals = [_eval_guard(g, cols) for (_, g, _) in _TRANS]
    zero = jnp.zeros((L, 1), dtype=p_block.dtype)

    rows = []
    for d in range(_N_STATES):
        row = [gvals[_EDGE[(d, s)]] if (d, s) in _EDGE else zero
               for s in range(_N_STATES)]
        rows.append(jnp.concatenate(row, axis=1)[:, None, :])
    return jnp.concatenate(rows, axis=1)


def _dot(x, y):
    return jax.lax.dot_general(
        x, y, (((1,), (0,)), ((), ())),
        precision=jax.lax.Precision.HIGHEST,
        preferred_element_type=jnp.float32)


def _fuzzy_kernel(p_ref, out_ref, a_ref, b_ref, c_ref):
    a_ref[:, :, :] = _build_mats(p_ref[:, :])

    for i in range(100):
        b_ref[i] = _dot(a_ref[2 * i + 1], a_ref[2 * i])
    for i in range(50):
        c_ref[i] = _dot(b_ref[2 * i + 1], b_ref[2 * i])
    mats = [_dot(c_ref[2 * i + 1], c_ref[2 * i]) for i in range(25)]
    while len(mats) > 1:
        nxt = [_dot(mats[2 * i + 1], mats[2 * i]) for i in range(len(mats) // 2)]
        if len(mats) % 2:
            nxt.append(mats[-1])
        mats = nxt

    out_ref[:, :] = mats[0]


def kernel(symbols_prob):
    out = pl.pallas_call(
        _fuzzy_kernel,
        out_shape=jax.ShapeDtypeStruct((_N_STATES, _N_STATES), symbols_prob.dtype),
        scratch_shapes=[
            pltpu.VMEM((_SEQ_LEN, _N_STATES, _N_STATES), symbols_prob.dtype),
            pltpu.VMEM((_SEQ_LEN // 2, _N_STATES, _N_STATES), symbols_prob.dtype),
            pltpu.VMEM((_SEQ_LEN // 4, _N_STATES, _N_STATES), symbols_prob.dtype),
        ],
    )(symbols_prob)
    return out[:, 0]
